# baseline (speedup 1.0000x reference)
import functools

import jax
import jax.numpy as jnp
from jax import lax
from jax.experimental import pallas as pl
from jax.experimental.pallas import tpu as pltpu
from jax.experimental.pallas import tpu_sc as plsc

N = 10000
E = 320000
D = 128

NC = 2
NS = 16
NW = NC * NS
CH = 128
EPW = 10240
NCH = EPW // CH
EPAD = NW * EPW
SINK = N
NP = 10240
RPS = NP // NS
ZR = 64

R = 1024
G = NP // R


def _fill(buf, nrows, value):
    v = jnp.full((16,), value, jnp.float32)

    def row(r, _):
        for k in range(D // 16):
            buf[r, pl.ds(k * 16, 16)] = v
        return 0

    lax.fori_loop(0, nrows, row, 0)


def _mesh():
    return plsc.VectorSubcoreMesh(core_axis_name="c", subcore_axis_name="s",
                                  num_cores=NC, num_subcores=NS)


def _zero_shared(sh, zbuf, sid):
    _fill(zbuf, ZR, 0.0)

    def zcopy(j, _):
        pltpu.sync_copy(zbuf, sh.at[pl.ds(sid * RPS + j * ZR, ZR)])
        return 0

    lax.fori_loop(0, RPS // ZR, zcopy, 0)


@functools.cache
def _deg_kernel():
    return pl.kernel(
        _deg_body,
        out_type=jax.ShapeDtypeStruct((NC, NP, D), jnp.float32),
        mesh=_mesh(),
        scratch_types=[
            pltpu.VMEM((NCH, CH), jnp.int32),
            pltpu.VMEM((CH,), jnp.int32),
            pltpu.VMEM((CH, D), jnp.float32),
            pltpu.VMEM((ZR, D), jnp.float32),
            pltpu.VMEM_SHARED((NP, D), jnp.float32),
        ],
    )


def _deg_body(dst_hbm, out_hbm, dst_all, dst_c, ones_b, zbuf, deg_sh):
    cid = lax.axis_index("c")
    sid = lax.axis_index("s")
    w = cid * NS + sid
    pltpu.sync_copy(dst_hbm.at[w], dst_all)
    _fill(ones_b, CH, 1.0)
    _zero_shared(deg_sh, zbuf, sid)
    plsc.subcore_barrier()

    def chunk(j, _):
        for k in range(CH // 16):
            dst_c[pl.ds(k * 16, 16)] = dst_all[j, pl.ds(k * 16, 16)]
        pltpu.sync_copy(ones_b, deg_sh.at[dst_c], add=True)
        return 0

    lax.fori_loop(0, NCH, chunk, 0)
    plsc.subcore_barrier()
    pltpu.sync_copy(deg_sh.at[pl.ds(sid * RPS, RPS)],
                    out_hbm.at[cid, pl.ds(sid * RPS, RPS)])


HNCH = NCH // 2


@functools.cache
def _seg_sum_kernel():
    return pl.kernel(
        _seg_sum_body,
        out_type=jax.ShapeDtypeStruct((NC, NP, D), jnp.float32),
        mesh=_mesh(),
        scratch_types=[
            pltpu.VMEM((NCH, CH), jnp.int32),
            pltpu.VMEM((NCH, CH), jnp.int32),
            pltpu.VMEM((CH,), jnp.int32),
            pltpu.VMEM((CH,), jnp.int32),
            pltpu.VMEM((CH, D), jnp.float32),
            pltpu.VMEM_SHARED((NP, D), jnp.float32),
            pltpu.SemaphoreType.DMA,
        ],
    )


def _seg_sum_body(g_hbm, src_hbm, dst_hbm, out_hbm,
                  src_tab, dst_tab, src_c, dst_c, rows, acc_sh, sem):
    cid = lax.axis_index("c")
    sid = lax.axis_index("s")
    w = cid * NS + sid
    pltpu.sync_copy(src_hbm.at[w], src_tab)
    pltpu.sync_copy(dst_hbm.at[w], dst_tab)
    _fill(rows, CH, 0.0)

    def zcopy(j, _):
        pltpu.sync_copy(rows, acc_sh.at[pl.ds(sid * RPS + j * CH, CH)])
        return 0

    lax.fori_loop(0, RPS // CH, zcopy, 0)
    plsc.subcore_barrier()

    def chunk(j, _):
        for k in range(CH // 16):
            src_c[pl.ds(k * 16, 16)] = src_tab[j, pl.ds(k * 16, 16)]
            dst_c[pl.ds(k * 16, 16)] = dst_tab[j, pl.ds(k * 16, 16)]
        pltpu.async_copy(g_hbm.at[src_c], rows, sem).wait()
        pltpu.sync_copy(rows, acc_sh.at[dst_c], add=True)
        return 0

    lax.fori_loop(0, NCH, chunk, 0)
    plsc.subcore_barrier()
    pltpu.sync_copy(acc_sh.at[pl.ds(sid * RPS, RPS)],
                    out_hbm.at[cid, pl.ds(sid * RPS, RPS)])


def _dis_block(deg_ref):
    return lax.rsqrt(deg_ref[0] + deg_ref[1] + 1.0)


def _tc1_body(x_ref, w_ref, deg_ref, g_ref):
    h = jnp.dot(x_ref[...], w_ref[...], preferred_element_type=jnp.float32)
    g_ref[...] = h * _dis_block(deg_ref)


def _tc2_body(acc_ref, g1_ref, w2_ref, b1_ref, deg_ref, g2_ref):
    dis = _dis_block(deg_ref)
    z = jnp.maximum(dis * (acc_ref[0] + acc_ref[1] + g1_ref[...]) + b1_ref[...], 0.0)
    h2 = jnp.dot(z, w2_ref[...], preferred_element_type=jnp.float32)
    g2_ref[...] = h2 * dis


def _tc3_body(acc_ref, g2_ref, b2_ref, deg_ref, z_ref, lsm_ref):
    dis = _dis_block(deg_ref)
    z = dis * (acc_ref[0] + acc_ref[1] + g2_ref[...]) + b2_ref[...]
    z_ref[...] = z
    m = jnp.max(z, axis=1, keepdims=True)
    lse = jnp.log(jnp.sum(jnp.exp(z - m), axis=1, keepdims=True)) + m
    lsm_ref[...] = z - lse


_row_spec = pl.BlockSpec((R, D), lambda i: (i, 0))
_acc_spec = pl.BlockSpec((NC, R, D), lambda i: (0, i, 0))
_w_spec = pl.BlockSpec((D, D), lambda i: (0, 0))
_b_spec = pl.BlockSpec((1, D), lambda i: (0, 0))

_tc1 = pl.pallas_call(
    _tc1_body,
    grid=(G,),
    in_specs=[_row_spec, _w_spec, _acc_spec],
    out_specs=_row_spec,
    out_shape=jax.ShapeDtypeStruct((NP, D), jnp.float32),
)

_tc2 = pl.pallas_call(
    _tc2_body,
    grid=(G,),
    in_specs=[_acc_spec, _row_spec, _w_spec, _b_spec, _acc_spec],
    out_specs=_row_spec,
    out_shape=jax.ShapeDtypeStruct((NP, D), jnp.float32),
)

_tc3 = pl.pallas_call(
    _tc3_body,
    grid=(G,),
    in_specs=[_acc_spec, _row_spec, _b_spec, _acc_spec],
    out_specs=(_row_spec, _row_spec),
    out_shape=(jax.ShapeDtypeStruct((NP, D), jnp.float32),
               jax.ShapeDtypeStruct((NP, D), jnp.float32)),
)


def kernel(x, edge_index, W1, b1, W2, b2):
    pad = EPAD - E
    src_r = jnp.concatenate(
        [edge_index[0], jnp.zeros((pad,), jnp.int32)]).reshape(NW, NCH, CH)
    dst_r = jnp.concatenate(
        [edge_index[1], jnp.full((pad,), SINK, jnp.int32)]).reshape(NW, NCH, CH)
    xp = jnp.pad(x, ((0, NP - N), (0, 0)))

    degp = _deg_kernel()(dst_r)
    g1 = _tc1(xp, W1, degp)
    acc1 = _seg_sum_kernel()(g1, src_r, dst_r)
    g2 = _tc2(acc1, g1, W2, b1.reshape(1, D), degp)
    acc2 = _seg_sum_kernel()(g2, src_r, dst_r)
    z2, lsm = _tc3(acc2, g2, b2.reshape(1, D), degp)
    return (z2[:N], lsm[:N])


# speedup vs baseline: 1.5056x; 1.5056x over previous
import functools

import jax
import jax.numpy as jnp
from jax import lax
from jax.experimental import pallas as pl
from jax.experimental.pallas import tpu as pltpu
from jax.experimental.pallas import tpu_sc as plsc

N = 10000
E = 320000
D = 128

NC = 2
NS = 16
NW = NC * NS
CH = 128
EPW = 10112
NCH = EPW // CH
EPAD = NW * EPW
SINK = N
NP = 10240
RPS = NP // NS
ZR = 64

R = 1024
G = NP // R


def _fill(buf, nrows, value):
    v = jnp.full((16,), value, jnp.float32)

    def row(r, _):
        for k in range(D // 16):
            buf[r, pl.ds(k * 16, 16)] = v
        return 0

    lax.fori_loop(0, nrows, row, 0)


def _mesh():
    return plsc.VectorSubcoreMesh(core_axis_name="c", subcore_axis_name="s",
                                  num_cores=NC, num_subcores=NS)


def _zero_shared(sh, zbuf, sid):
    _fill(zbuf, ZR, 0.0)

    def zcopy(j, _):
        pltpu.sync_copy(zbuf, sh.at[pl.ds(sid * RPS + j * ZR, ZR)])
        return 0

    lax.fori_loop(0, RPS // ZR, zcopy, 0)


@functools.cache
def _deg_kernel():
    return pl.kernel(
        _deg_body,
        out_type=jax.ShapeDtypeStruct((NC, NP, D), jnp.float32),
        mesh=_mesh(),
        scratch_types=[
            pltpu.VMEM((NCH, CH), jnp.int32),
            pltpu.VMEM((CH,), jnp.int32),
            pltpu.VMEM((CH, D), jnp.float32),
            pltpu.VMEM((ZR, D), jnp.float32),
            pltpu.VMEM_SHARED((NP, D), jnp.float32),
        ],
    )


def _deg_body(dst_hbm, out_hbm, dst_all, dst_c, ones_b, zbuf, deg_sh):
    cid = lax.axis_index("c")
    sid = lax.axis_index("s")
    w = cid * NS + sid
    pltpu.sync_copy(dst_hbm.at[w], dst_all)
    _fill(ones_b, CH, 1.0)
    _zero_shared(deg_sh, zbuf, sid)
    plsc.subcore_barrier()

    def chunk(j, _):
        for k in range(CH // 16):
            dst_c[pl.ds(k * 16, 16)] = dst_all[j, pl.ds(k * 16, 16)]
        pltpu.sync_copy(ones_b, deg_sh.at[dst_c], add=True)
        return 0

    lax.fori_loop(0, NCH, chunk, 0)
    plsc.subcore_barrier()
    pltpu.sync_copy(deg_sh.at[pl.ds(sid * RPS, RPS)],
                    out_hbm.at[cid, pl.ds(sid * RPS, RPS)])


@functools.cache
def _seg_sum_kernel():
    return pl.kernel(
        _seg_sum_body,
        out_type=jax.ShapeDtypeStruct((NC, NP, D), jnp.float32),
        mesh=_mesh(),
        scratch_types=[
            pltpu.VMEM((NCH, CH), jnp.int32),
            pltpu.VMEM((NCH, CH), jnp.int32),
            pltpu.VMEM((CH,), jnp.int32),
            pltpu.VMEM((CH,), jnp.int32),
            pltpu.VMEM((CH, D), jnp.float32),
            pltpu.VMEM((ZR, D), jnp.float32),
            pltpu.VMEM_SHARED((NP, D), jnp.float32),
            pltpu.SemaphoreType.DMA,
        ],
    )


def _seg_sum_body(g_hbm, src_hbm, dst_hbm, out_hbm,
                  src_all, dst_all, src_c, dst_c, rows, zbuf, acc_sh, sem):
    cid = lax.axis_index("c")
    sid = lax.axis_index("s")
    w = cid * NS + sid
    pltpu.sync_copy(src_hbm.at[w], src_all)
    pltpu.sync_copy(dst_hbm.at[w], dst_all)
    _zero_shared(acc_sh, zbuf, sid)
    plsc.subcore_barrier()

    def chunk(j, _):
        for k in range(CH // 16):
            src_c[pl.ds(k * 16, 16)] = src_all[j, pl.ds(k * 16, 16)]
            dst_c[pl.ds(k * 16, 16)] = dst_all[j, pl.ds(k * 16, 16)]
        pltpu.async_copy(g_hbm.at[src_c], rows, sem).wait()
        pltpu.sync_copy(rows, acc_sh.at[dst_c], add=True)
        return 0

    lax.fori_loop(0, NCH, chunk, 0)
    plsc.subcore_barrier()
    pltpu.sync_copy(acc_sh.at[pl.ds(sid * RPS, RPS)],
                    out_hbm.at[cid, pl.ds(sid * RPS, RPS)])


def _dis_block(deg_ref):
    return lax.rsqrt(deg_ref[0] + deg_ref[1] + 1.0)


def _tc1_body(x_ref, w_ref, deg_ref, g_ref):
    h = jnp.dot(x_ref[...], w_ref[...], preferred_element_type=jnp.float32)
    g_ref[...] = h * _dis_block(deg_ref)


def _tc2_body(acc_ref, g1_ref, w2_ref, b1_ref, deg_ref, g2_ref):
    dis = _dis_block(deg_ref)
    z = jnp.maximum(dis * (acc_ref[0] + acc_ref[1] + g1_ref[...]) + b1_ref[...], 0.0)
    h2 = jnp.dot(z, w2_ref[...], preferred_element_type=jnp.float32)
    g2_ref[...] = h2 * dis


def _tc3_body(acc_ref, g2_ref, b2_ref, deg_ref, z_ref, lsm_ref):
    dis = _dis_block(deg_ref)
    z = dis * (acc_ref[0] + acc_ref[1] + g2_ref[...]) + b2_ref[...]
    z_ref[...] = z
    m = jnp.max(z, axis=1, keepdims=True)
    lse = jnp.log(jnp.sum(jnp.exp(z - m), axis=1, keepdims=True)) + m
    lsm_ref[...] = z - lse


_row_spec = pl.BlockSpec((R, D), lambda i: (i, 0))
_acc_spec = pl.BlockSpec((NC, R, D), lambda i: (0, i, 0))
_w_spec = pl.BlockSpec((D, D), lambda i: (0, 0))
_b_spec = pl.BlockSpec((1, D), lambda i: (0, 0))

_tc1 = pl.pallas_call(
    _tc1_body,
    grid=(G,),
    in_specs=[_row_spec, _w_spec, _acc_spec],
    out_specs=_row_spec,
    out_shape=jax.ShapeDtypeStruct((NP, D), jnp.float32),
)

_tc2 = pl.pallas_call(
    _tc2_body,
    grid=(G,),
    in_specs=[_acc_spec, _row_spec, _w_spec, _b_spec, _acc_spec],
    out_specs=_row_spec,
    out_shape=jax.ShapeDtypeStruct((NP, D), jnp.float32),
)

_tc3 = pl.pallas_call(
    _tc3_body,
    grid=(G,),
    in_specs=[_acc_spec, _row_spec, _b_spec, _acc_spec],
    out_specs=(_row_spec, _row_spec),
    out_shape=(jax.ShapeDtypeStruct((NP, D), jnp.float32),
               jax.ShapeDtypeStruct((NP, D), jnp.float32)),
)


def kernel(x, edge_index, W1, b1, W2, b2):
    pad = EPAD - E
    src_r = jnp.concatenate(
        [edge_index[0], jnp.zeros((pad,), jnp.int32)]).reshape(NW, NCH, CH)
    dst_r = jnp.concatenate(
        [edge_index[1], jnp.full((pad,), SINK, jnp.int32)]).reshape(NW, NCH, CH)
    xp = jnp.pad(x, ((0, NP - N), (0, 0)))

    degp = _deg_kernel()(dst_r)
    g1 = _tc1(xp, W1, degp)
    acc1 = _seg_sum_kernel()(g1, src_r, dst_r)
    g2 = _tc2(acc1, g1, W2, b1.reshape(1, D), degp)
    acc2 = _seg_sum_kernel()(g2, src_r, dst_r)
    z2, lsm = _tc3(acc2, g2, b2.reshape(1, D), degp)
    return (z2[:N], lsm[:N])


# speedup vs baseline: 1.8556x; 1.2325x over previous
import functools

import jax
import jax.numpy as jnp
from jax import lax
from jax.experimental import pallas as pl
from jax.experimental.pallas import tpu as pltpu
from jax.experimental.pallas import tpu_sc as plsc

N = 10000
E = 320000
D = 128

NC = 2
NS = 16
NW = NC * NS
CH = 128
EPW = 10112
NCH = EPW // CH
EPAD = NW * EPW
SINK = N
NP = 10240
RPS = NP // NS
ZR = 64

R = 1024
G = NP // R


def _fill(buf, nrows, value):
    v = jnp.full((16,), value, jnp.float32)

    def row(r, _):
        for k in range(D // 16):
            buf[r, pl.ds(k * 16, 16)] = v
        return 0

    lax.fori_loop(0, nrows, row, 0)


def _mesh():
    return plsc.VectorSubcoreMesh(core_axis_name="c", subcore_axis_name="s",
                                  num_cores=NC, num_subcores=NS)


def _zero_shared(sh, zbuf, sid):
    _fill(zbuf, ZR, 0.0)

    def zcopy(j, _):
        pltpu.sync_copy(zbuf, sh.at[pl.ds(sid * RPS + j * ZR, ZR)])
        return 0

    lax.fori_loop(0, RPS // ZR, zcopy, 0)


@functools.cache
def _deg_kernel():
    return pl.kernel(
        _deg_body,
        out_type=jax.ShapeDtypeStruct((NC, NP, D), jnp.float32),
        mesh=_mesh(),
        scratch_types=[
            pltpu.VMEM((NCH, CH), jnp.int32),
            pltpu.VMEM((CH,), jnp.int32),
            pltpu.VMEM((CH, D), jnp.float32),
            pltpu.VMEM((ZR, D), jnp.float32),
            pltpu.VMEM_SHARED((NP, D), jnp.float32),
        ],
    )


def _deg_body(dst_hbm, out_hbm, dst_all, dst_c, ones_b, zbuf, deg_sh):
    cid = lax.axis_index("c")
    sid = lax.axis_index("s")
    w = cid * NS + sid
    pltpu.sync_copy(dst_hbm.at[w], dst_all)
    _fill(ones_b, CH, 1.0)
    _zero_shared(deg_sh, zbuf, sid)
    plsc.subcore_barrier()

    def chunk(j, _):
        for k in range(CH // 16):
            dst_c[pl.ds(k * 16, 16)] = dst_all[j, pl.ds(k * 16, 16)]
        pltpu.sync_copy(ones_b, deg_sh.at[dst_c], add=True)
        return 0

    lax.fori_loop(0, NCH, chunk, 0)
    plsc.subcore_barrier()
    pltpu.sync_copy(deg_sh.at[pl.ds(sid * RPS, RPS)],
                    out_hbm.at[cid, pl.ds(sid * RPS, RPS)])


C0 = 53
C1 = 104
CM = max(C0, C1)


@functools.cache
def _seg_sum_kernel():
    return pl.kernel(
        _seg_sum_body,
        out_type=jax.ShapeDtypeStruct((NC, NP, D), jnp.float32),
        mesh=_mesh(),
        scratch_types=[
            pltpu.VMEM((CM, CH), jnp.int32),
            pltpu.VMEM((CM, CH), jnp.int32),
            pltpu.VMEM((CH,), jnp.int32),
            pltpu.VMEM((CH,), jnp.int32),
            pltpu.VMEM((CH, D), jnp.float32),
            pltpu.VMEM_SHARED((NP, D), jnp.float32),
            pltpu.SemaphoreType.DMA,
        ],
    )


def _seg_sum_body(g_hbm, src_hbm, dst_hbm, out_hbm,
                  src_all, dst_all, src_c, dst_c, rows, acc_sh, sem):
    cid = lax.axis_index("c")
    sid = lax.axis_index("s")
    w = cid * NS + sid
    pltpu.sync_copy(src_hbm.at[w], src_all)
    pltpu.sync_copy(dst_hbm.at[w], dst_all)
    _fill(rows, CH, 0.0)

    def zcopy(j, _):
        pltpu.sync_copy(rows, acc_sh.at[pl.ds(sid * RPS + j * CH, CH)])
        return 0

    lax.fori_loop(0, RPS // CH, zcopy, 0)
    plsc.subcore_barrier()

    def chunk(j, _):
        for k in range(CH // 16):
            src_c[pl.ds(k * 16, 16)] = src_all[j, pl.ds(k * 16, 16)]
            dst_c[pl.ds(k * 16, 16)] = dst_all[j, pl.ds(k * 16, 16)]
        pltpu.async_copy(g_hbm.at[src_c], rows, sem).wait()
        pltpu.sync_copy(rows, acc_sh.at[dst_c], add=True)
        return 0

    nch = jnp.where(cid == 0, C0, C1)
    lax.fori_loop(0, nch, chunk, 0)
    plsc.subcore_barrier()
    pltpu.sync_copy(acc_sh.at[pl.ds(sid * RPS, RPS)],
                    out_hbm.at[cid, pl.ds(sid * RPS, RPS)])


def _dis_block(deg_ref):
    return lax.rsqrt(deg_ref[0] + deg_ref[1] + 1.0)


def _tc1_body(x_ref, w_ref, deg_ref, g_ref):
    h = jnp.dot(x_ref[...], w_ref[...], preferred_element_type=jnp.float32)
    g_ref[...] = h * _dis_block(deg_ref)


def _tc2_body(acc_ref, g1_ref, w2_ref, b1_ref, deg_ref, g2_ref):
    dis = _dis_block(deg_ref)
    z = jnp.maximum(dis * (acc_ref[0] + acc_ref[1] + g1_ref[...]) + b1_ref[...], 0.0)
    h2 = jnp.dot(z, w2_ref[...], preferred_element_type=jnp.float32)
    g2_ref[...] = h2 * dis


def _tc3_body(acc_ref, g2_ref, b2_ref, deg_ref, z_ref, lsm_ref):
    dis = _dis_block(deg_ref)
    z = dis * (acc_ref[0] + acc_ref[1] + g2_ref[...]) + b2_ref[...]
    z_ref[...] = z
    m = jnp.max(z, axis=1, keepdims=True)
    lse = jnp.log(jnp.sum(jnp.exp(z - m), axis=1, keepdims=True)) + m
    lsm_ref[...] = z - lse


_row_spec = pl.BlockSpec((R, D), lambda i: (i, 0))
_acc_spec = pl.BlockSpec((NC, R, D), lambda i: (0, i, 0))
_w_spec = pl.BlockSpec((D, D), lambda i: (0, 0))
_b_spec = pl.BlockSpec((1, D), lambda i: (0, 0))

_tc1 = pl.pallas_call(
    _tc1_body,
    grid=(G,),
    in_specs=[_row_spec, _w_spec, _acc_spec],
    out_specs=_row_spec,
    out_shape=jax.ShapeDtypeStruct((NP, D), jnp.float32),
)

_tc2 = pl.pallas_call(
    _tc2_body,
    grid=(G,),
    in_specs=[_acc_spec, _row_spec, _w_spec, _b_spec, _acc_spec],
    out_specs=_row_spec,
    out_shape=jax.ShapeDtypeStruct((NP, D), jnp.float32),
)

_tc3 = pl.pallas_call(
    _tc3_body,
    grid=(G,),
    in_specs=[_acc_spec, _row_spec, _b_spec, _acc_spec],
    out_specs=(_row_spec, _row_spec),
    out_shape=(jax.ShapeDtypeStruct((NP, D), jnp.float32),
               jax.ShapeDtypeStruct((NP, D), jnp.float32)),
)


def _skew_split(arr, fill):
    asz = NS * C0 * CH
    blk0 = jnp.pad(arr[:asz].reshape(NS, C0, CH),
                   ((0, 0), (0, CM - C0), (0, 0)), constant_values=fill)
    per = (E - asz) // NS
    blk1 = jnp.pad(arr[asz:].reshape(NS, per),
                   ((0, 0), (0, C1 * CH - per)), constant_values=fill)
    blk1 = jnp.pad(blk1.reshape(NS, C1, CH),
                   ((0, 0), (0, CM - C1), (0, 0)), constant_values=fill)
    return jnp.concatenate([blk0, blk1])


def kernel(x, edge_index, W1, b1, W2, b2):
    pad = EPAD - E
    dst_r = jnp.concatenate(
        [edge_index[1], jnp.full((pad,), SINK, jnp.int32)]).reshape(NW, NCH, CH)
    src_s = _skew_split(edge_index[0], 0)
    dst_s = _skew_split(edge_index[1], SINK)
    xp = jnp.pad(x, ((0, NP - N), (0, 0)))

    degp = _deg_kernel()(dst_r)
    g1 = _tc1(xp, W1, degp)
    acc1 = _seg_sum_kernel()(g1, src_s, dst_s)
    g2 = _tc2(acc1, g1, W2, b1.reshape(1, D), degp)
    acc2 = _seg_sum_kernel()(g2, src_s, dst_s)
    z2, lsm = _tc3(acc2, g2, b2.reshape(1, D), degp)
    return (z2[:N], lsm[:N])


# speedup vs baseline: 1.9428x; 1.0470x over previous
import functools

import jax
import jax.numpy as jnp
from jax import lax
from jax.experimental import pallas as pl
from jax.experimental.pallas import tpu as pltpu
from jax.experimental.pallas import tpu_sc as plsc

N = 10000
E = 320000
D = 128

NC = 2
NS = 16
NW = NC * NS
CH = 128
EPW = 10112
NCH = EPW // CH
EPAD = NW * EPW
SINK = N
NP = 10240
RPS = NP // NS
ZR = 64

R = 1024
G = NP // R


def _fill(buf, nrows, value):
    v = jnp.full((16,), value, jnp.float32)

    def row(r, _):
        for k in range(D // 16):
            buf[r, pl.ds(k * 16, 16)] = v
        return 0

    lax.fori_loop(0, nrows, row, 0)


def _mesh():
    return plsc.VectorSubcoreMesh(core_axis_name="c", subcore_axis_name="s",
                                  num_cores=NC, num_subcores=NS)


def _zero_shared(sh, zbuf, sid):
    _fill(zbuf, ZR, 0.0)

    def zcopy(j, _):
        pltpu.sync_copy(zbuf, sh.at[pl.ds(sid * RPS + j * ZR, ZR)])
        return 0

    lax.fori_loop(0, RPS // ZR, zcopy, 0)


@functools.cache
def _deg_kernel():
    return pl.kernel(
        _deg_body,
        out_type=jax.ShapeDtypeStruct((NC, NP, D), jnp.float32),
        mesh=_mesh(),
        scratch_types=[
            pltpu.VMEM((NCH, CH), jnp.int32),
            pltpu.VMEM((CH,), jnp.int32),
            pltpu.VMEM((CH, D), jnp.float32),
            pltpu.VMEM((ZR, D), jnp.float32),
            pltpu.VMEM_SHARED((NP, D), jnp.float32),
        ],
    )


def _deg_body(dst_hbm, out_hbm, dst_all, dst_c, ones_b, zbuf, deg_sh):
    cid = lax.axis_index("c")
    sid = lax.axis_index("s")
    w = cid * NS + sid
    pltpu.sync_copy(dst_hbm.at[w], dst_all)
    _fill(ones_b, CH, 1.0)
    _zero_shared(deg_sh, zbuf, sid)
    plsc.subcore_barrier()

    def chunk(j, _):
        for k in range(CH // 16):
            dst_c[pl.ds(k * 16, 16)] = dst_all[j, pl.ds(k * 16, 16)]
        pltpu.sync_copy(ones_b, deg_sh.at[dst_c], add=True)
        return 0

    lax.fori_loop(0, NCH, chunk, 0)
    plsc.subcore_barrier()
    pltpu.sync_copy(deg_sh.at[pl.ds(sid * RPS, RPS)],
                    out_hbm.at[cid, pl.ds(sid * RPS, RPS)])


C0 = 104
C1 = 53
CM = max(C0, C1)


@functools.cache
def _seg_sum_kernel():
    return pl.kernel(
        _seg_sum_body,
        out_type=jax.ShapeDtypeStruct((NC, NP, D), jnp.float32),
        mesh=_mesh(),
        scratch_types=[
            pltpu.VMEM((CM, CH), jnp.int32),
            pltpu.VMEM((CM, CH), jnp.int32),
            pltpu.VMEM((CH,), jnp.int32),
            pltpu.VMEM((CH,), jnp.int32),
            pltpu.VMEM((CH, D), jnp.float32),
            pltpu.VMEM_SHARED((NP, D), jnp.float32),
            pltpu.SemaphoreType.DMA,
        ],
    )


def _seg_sum_body(g_hbm, src_hbm, dst_hbm, out_hbm,
                  src_all, dst_all, src_c, dst_c, rows, acc_sh, sem):
    cid = lax.axis_index("c")
    sid = lax.axis_index("s")
    w = cid * NS + sid
    pltpu.sync_copy(src_hbm.at[w], src_all)
    pltpu.sync_copy(dst_hbm.at[w], dst_all)
    _fill(rows, CH, 0.0)

    def zcopy(j, _):
        pltpu.sync_copy(rows, acc_sh.at[pl.ds(sid * RPS + j * CH, CH)])
        return 0

    lax.fori_loop(0, RPS // CH, zcopy, 0)
    plsc.subcore_barrier()

    def chunk(j, _):
        for k in range(CH // 16):
            src_c[pl.ds(k * 16, 16)] = src_all[j, pl.ds(k * 16, 16)]
            dst_c[pl.ds(k * 16, 16)] = dst_all[j, pl.ds(k * 16, 16)]
        pltpu.async_copy(g_hbm.at[src_c], rows, sem).wait()
        pltpu.sync_copy(rows, acc_sh.at[dst_c], add=True)
        return 0

    nch = jnp.where(cid == 0, C0, C1)
    lax.fori_loop(0, nch, chunk, 0)
    plsc.subcore_barrier()
    pltpu.sync_copy(acc_sh.at[pl.ds(sid * RPS, RPS)],
                    out_hbm.at[cid, pl.ds(sid * RPS, RPS)])


def _dis_block(deg_ref):
    return lax.rsqrt(deg_ref[0] + deg_ref[1] + 1.0)


def _tc1_body(x_ref, w_ref, deg_ref, g_ref):
    h = jnp.dot(x_ref[...], w_ref[...], preferred_element_type=jnp.float32)
    g_ref[...] = h * _dis_block(deg_ref)


def _tc2_body(acc_ref, g1_ref, w2_ref, b1_ref, deg_ref, g2_ref):
    dis = _dis_block(deg_ref)
    z = jnp.maximum(dis * (acc_ref[0] + acc_ref[1] + g1_ref[...]) + b1_ref[...], 0.0)
    h2 = jnp.dot(z, w2_ref[...], preferred_element_type=jnp.float32)
    g2_ref[...] = h2 * dis


def _tc3_body(acc_ref, g2_ref, b2_ref, deg_ref, z_ref, lsm_ref):
    dis = _dis_block(deg_ref)
    z = dis * (acc_ref[0] + acc_ref[1] + g2_ref[...]) + b2_ref[...]
    z_ref[...] = z
    m = jnp.max(z, axis=1, keepdims=True)
    lse = jnp.log(jnp.sum(jnp.exp(z - m), axis=1, keepdims=True)) + m
    lsm_ref[...] = z - lse


_row_spec = pl.BlockSpec((R, D), lambda i: (i, 0))
_acc_spec = pl.BlockSpec((NC, R, D), lambda i: (0, i, 0))
_w_spec = pl.BlockSpec((D, D), lambda i: (0, 0))
_b_spec = pl.BlockSpec((1, D), lambda i: (0, 0))

_tc1 = pl.pallas_call(
    _tc1_body,
    grid=(G,),
    in_specs=[_row_spec, _w_spec, _acc_spec],
    out_specs=_row_spec,
    out_shape=jax.ShapeDtypeStruct((NP, D), jnp.float32),
)

_tc2 = pl.pallas_call(
    _tc2_body,
    grid=(G,),
    in_specs=[_acc_spec, _row_spec, _w_spec, _b_spec, _acc_spec],
    out_specs=_row_spec,
    out_shape=jax.ShapeDtypeStruct((NP, D), jnp.float32),
)

_tc3 = pl.pallas_call(
    _tc3_body,
    grid=(G,),
    in_specs=[_acc_spec, _row_spec, _b_spec, _acc_spec],
    out_specs=(_row_spec, _row_spec),
    out_shape=(jax.ShapeDtypeStruct((NP, D), jnp.float32),
               jax.ShapeDtypeStruct((NP, D), jnp.float32)),
)


def _skew_split(arr, fill):
    asz = NS * C0 * CH
    blk0 = jnp.pad(arr[:asz].reshape(NS, C0, CH),
                   ((0, 0), (0, CM - C0), (0, 0)), constant_values=fill)
    per = (E - asz) // NS
    blk1 = jnp.pad(arr[asz:].reshape(NS, per),
                   ((0, 0), (0, C1 * CH - per)), constant_values=fill)
    blk1 = jnp.pad(blk1.reshape(NS, C1, CH),
                   ((0, 0), (0, CM - C1), (0, 0)), constant_values=fill)
    return jnp.concatenate([blk0, blk1])


def kernel(x, edge_index, W1, b1, W2, b2):
    pad = EPAD - E
    dst_r = jnp.concatenate(
        [edge_index[1], jnp.full((pad,), SINK, jnp.int32)]).reshape(NW, NCH, CH)
    src_s = _skew_split(edge_index[0], 0)
    dst_s = _skew_split(edge_index[1], SINK)
    xp = jnp.pad(x, ((0, NP - N), (0, 0)))

    degp = _deg_kernel()(dst_r)
    g1 = _tc1(xp, W1, degp)
    acc1 = _seg_sum_kernel()(g1, src_s, dst_s)
    g2 = _tc2(acc1, g1, W2, b1.reshape(1, D), degp)
    acc2 = _seg_sum_kernel()(g2, src_s, dst_s)
    z2, lsm = _tc3(acc2, g2, b2.reshape(1, D), degp)
    return (z2[:N], lsm[:N])


# speedup vs baseline: 2.1325x; 1.0976x over previous
import functools

import jax
import jax.numpy as jnp
from jax import lax
from jax.experimental import pallas as pl
from jax.experimental.pallas import tpu as pltpu
from jax.experimental.pallas import tpu_sc as plsc

N = 10000
E = 320000
D = 128

NC = 2
NS = 16
NW = NC * NS
CH = 128
EPW = 10112
NCH = EPW // CH
EPAD = NW * EPW
SINK = N
NP = 10240
RPS = NP // NS
ZR = 64

R = 1024
G = NP // R


def _fill(buf, nrows, value):
    v = jnp.full((16,), value, jnp.float32)

    def row(r, _):
        for k in range(D // 16):
            buf[r, pl.ds(k * 16, 16)] = v
        return 0

    lax.fori_loop(0, nrows, row, 0)


def _mesh():
    return plsc.VectorSubcoreMesh(core_axis_name="c", subcore_axis_name="s",
                                  num_cores=NC, num_subcores=NS)


def _zero_shared(sh, zbuf, sid):
    _fill(zbuf, ZR, 0.0)

    def zcopy(j, _):
        pltpu.sync_copy(zbuf, sh.at[pl.ds(sid * RPS + j * ZR, ZR)])
        return 0

    lax.fori_loop(0, RPS // ZR, zcopy, 0)


@functools.cache
def _deg_kernel():
    return pl.kernel(
        _deg_body,
        out_type=jax.ShapeDtypeStruct((NC, NP, D), jnp.float32),
        mesh=_mesh(),
        scratch_types=[
            pltpu.VMEM((NCH, CH), jnp.int32),
            pltpu.VMEM((CH,), jnp.int32),
            pltpu.VMEM((CH,), jnp.int32),
            pltpu.VMEM((CH, D), jnp.float32),
            pltpu.VMEM((ZR, D), jnp.float32),
            pltpu.VMEM_SHARED((NP, D), jnp.float32),
        ],
    )


def _deg_body(dst_hbm, out_hbm, dst_all, dst_c0, dst_c1, ones_b, zbuf, deg_sh):
    cid = lax.axis_index("c")
    sid = lax.axis_index("s")
    w = cid * NS + sid
    _fill(ones_b, CH, 1.0)
    _fill(zbuf, ZR, 0.0)
    pltpu.sync_copy(dst_hbm.at[w], dst_all)

    def zcopy(j, _):
        pltpu.sync_copy(zbuf, deg_sh.at[pl.ds(sid * RPS + j * ZR, ZR)])
        return 0

    lax.fori_loop(0, RPS // ZR, zcopy, 0)

    def stage(j, dst_c):
        for k in range(CH // 16):
            dst_c[pl.ds(k * 16, 16)] = dst_all[j, pl.ds(k * 16, 16)]

    stage(0, dst_c0)
    plsc.subcore_barrier()

    def pair(i, _):
        j = 2 * i

        @pl.when(j < NCH)
        def _():
            stage(j + 1, dst_c1)
            pltpu.sync_copy(ones_b, deg_sh.at[dst_c0], add=True)

        @pl.when(j + 1 < NCH)
        def _():
            stage(j + 2, dst_c0)
            pltpu.sync_copy(ones_b, deg_sh.at[dst_c1], add=True)

        return 0

    lax.fori_loop(0, (NCH + 1) // 2, pair, 0)
    plsc.subcore_barrier()
    pltpu.sync_copy(deg_sh.at[pl.ds(sid * RPS, RPS)],
                    out_hbm.at[cid, pl.ds(sid * RPS, RPS)])


C0 = 93
C1 = 64
CM = max(C0, C1)


@functools.cache
def _seg_sum_kernel():
    return pl.kernel(
        _seg_sum_body,
        out_type=jax.ShapeDtypeStruct((NC, NP, D), jnp.float32),
        mesh=_mesh(),
        scratch_types=[
            pltpu.VMEM((CM, CH), jnp.int32),
            pltpu.VMEM((CM, CH), jnp.int32),
            pltpu.VMEM((CH,), jnp.int32),
            pltpu.VMEM((CH,), jnp.int32),
            pltpu.VMEM((CH,), jnp.int32),
            pltpu.VMEM((CH,), jnp.int32),
            pltpu.VMEM((CH, D), jnp.float32),
            pltpu.VMEM_SHARED((NP, D), jnp.float32),
            pltpu.SemaphoreType.DMA,
        ],
    )


def _seg_sum_body(g_hbm, src_hbm, dst_hbm, out_hbm,
                  src_all, dst_all, src_c0, src_c1, dst_c0, dst_c1,
                  rows, acc_sh, sem):
    cid = lax.axis_index("c")
    sid = lax.axis_index("s")
    w = cid * NS + sid
    _fill(rows, CH, 0.0)
    pltpu.sync_copy(src_hbm.at[w], src_all)
    pltpu.sync_copy(dst_hbm.at[w], dst_all)

    def zcopy(j, _):
        pltpu.sync_copy(rows, acc_sh.at[pl.ds(sid * RPS + j * CH, CH)])
        return 0

    lax.fori_loop(0, RPS // CH, zcopy, 0)

    def stage(j, src_c, dst_c):
        for k in range(CH // 16):
            src_c[pl.ds(k * 16, 16)] = src_all[j, pl.ds(k * 16, 16)]
            dst_c[pl.ds(k * 16, 16)] = dst_all[j, pl.ds(k * 16, 16)]

    stage(0, src_c0, dst_c0)
    plsc.subcore_barrier()

    nch = jnp.where(cid == 0, C0, C1)

    def pair(i, _):
        j = 2 * i
        @pl.when(j < nch)
        def _():
            pltpu.async_copy(g_hbm.at[src_c0], rows, sem)
            stage(j + 1, src_c1, dst_c1)
            pltpu.make_async_copy(g_hbm.at[src_c0], rows, sem).wait()
            pltpu.sync_copy(rows, acc_sh.at[dst_c0], add=True)

        @pl.when(j + 1 < nch)
        def _():
            pltpu.async_copy(g_hbm.at[src_c1], rows, sem)
            stage(j + 2, src_c0, dst_c0)
            pltpu.make_async_copy(g_hbm.at[src_c1], rows, sem).wait()
            pltpu.sync_copy(rows, acc_sh.at[dst_c1], add=True)

        return 0

    lax.fori_loop(0, (CM + 2) // 2, pair, 0)
    plsc.subcore_barrier()
    pltpu.sync_copy(acc_sh.at[pl.ds(sid * RPS, RPS)],
                    out_hbm.at[cid, pl.ds(sid * RPS, RPS)])


def _dis_block(deg_ref):
    return lax.rsqrt(deg_ref[0] + deg_ref[1] + 1.0)


def _tc1_body(x_ref, w_ref, deg_ref, g_ref):
    h = jnp.dot(x_ref[...], w_ref[...], preferred_element_type=jnp.float32)
    g_ref[...] = h * _dis_block(deg_ref)


def _tc2_body(acc_ref, g1_ref, w2_ref, b1_ref, deg_ref, g2_ref):
    dis = _dis_block(deg_ref)
    z = jnp.maximum(dis * (acc_ref[0] + acc_ref[1] + g1_ref[...]) + b1_ref[...], 0.0)
    h2 = jnp.dot(z, w2_ref[...], preferred_element_type=jnp.float32)
    g2_ref[...] = h2 * dis


def _tc3_body(acc_ref, g2_ref, b2_ref, deg_ref, z_ref, lsm_ref):
    dis = _dis_block(deg_ref)
    z = dis * (acc_ref[0] + acc_ref[1] + g2_ref[...]) + b2_ref[...]
    z_ref[...] = z
    m = jnp.max(z, axis=1, keepdims=True)
    lse = jnp.log(jnp.sum(jnp.exp(z - m), axis=1, keepdims=True)) + m
    lsm_ref[...] = z - lse


_row_spec = pl.BlockSpec((R, D), lambda i: (i, 0))
_acc_spec = pl.BlockSpec((NC, R, D), lambda i: (0, i, 0))
_w_spec = pl.BlockSpec((D, D), lambda i: (0, 0))
_b_spec = pl.BlockSpec((1, D), lambda i: (0, 0))

_tc1 = pl.pallas_call(
    _tc1_body,
    grid=(G,),
    in_specs=[_row_spec, _w_spec, _acc_spec],
    out_specs=_row_spec,
    out_shape=jax.ShapeDtypeStruct((NP, D), jnp.float32),
)

_tc2 = pl.pallas_call(
    _tc2_body,
    grid=(G,),
    in_specs=[_acc_spec, _row_spec, _w_spec, _b_spec, _acc_spec],
    out_specs=_row_spec,
    out_shape=jax.ShapeDtypeStruct((NP, D), jnp.float32),
)

_tc3 = pl.pallas_call(
    _tc3_body,
    grid=(G,),
    in_specs=[_acc_spec, _row_spec, _b_spec, _acc_spec],
    out_specs=(_row_spec, _row_spec),
    out_shape=(jax.ShapeDtypeStruct((NP, D), jnp.float32),
               jax.ShapeDtypeStruct((NP, D), jnp.float32)),
)


def _skew_split(arr, fill):
    asz = NS * C0 * CH
    blk0 = jnp.pad(arr[:asz].reshape(NS, C0, CH),
                   ((0, 0), (0, CM - C0), (0, 0)), constant_values=fill)
    per = (E - asz) // NS
    blk1 = jnp.pad(arr[asz:].reshape(NS, per),
                   ((0, 0), (0, C1 * CH - per)), constant_values=fill)
    blk1 = jnp.pad(blk1.reshape(NS, C1, CH),
                   ((0, 0), (0, CM - C1), (0, 0)), constant_values=fill)
    return jnp.concatenate([blk0, blk1])


def kernel(x, edge_index, W1, b1, W2, b2):
    pad = EPAD - E
    dst_r = jnp.concatenate(
        [edge_index[1], jnp.full((pad,), SINK, jnp.int32)]).reshape(NW, NCH, CH)
    src_s = _skew_split(edge_index[0], 0)
    dst_s = _skew_split(edge_index[1], SINK)
    xp = jnp.pad(x, ((0, NP - N), (0, 0)))

    degp = _deg_kernel()(dst_r)
    g1 = _tc1(xp, W1, degp)
    acc1 = _seg_sum_kernel()(g1, src_s, dst_s)
    g2 = _tc2(acc1, g1, W2, b1.reshape(1, D), degp)
    acc2 = _seg_sum_kernel()(g2, src_s, dst_s)
    z2, lsm = _tc3(acc2, g2, b2.reshape(1, D), degp)
    return (z2[:N], lsm[:N])
